# baseline (speedup 1.0000x reference)
import functools

import jax
import jax.numpy as jnp
from jax.experimental import pallas as pl
from jax.experimental.pallas import tpu as pltpu

D = 256
TEMPERATURE = 10.0
BLOCK = 512
OUT_W = 16


def _prep_kernel(a1, a2, b1, b2, bvi, bot, bvt, boi,
                 gwc, gbc, wgc, gws, gbs, wgs,
                 mt, mi, bt, bi, uc, cc, us, cs):
    f32 = jnp.float32
    mt[:] = jnp.dot(a1[:], a2[:], preferred_element_type=f32)
    mi[:] = jnp.dot(b1[:], b2[:], preferred_element_type=f32)
    bt[:] = jnp.dot(bvi[:], a2[:], preferred_element_type=f32) + bot[:]
    bi[:] = jnp.dot(bvt[:], b2[:], preferred_element_type=f32) + boi[:]
    uc[:] = jnp.dot(gwc[:], wgc[:], preferred_element_type=f32)
    cc[:] = jnp.dot(gbc[:], wgc[:], preferred_element_type=f32)
    us[:] = jnp.dot(gws[:], wgs[:], preferred_element_type=f32)
    cs[:] = jnp.dot(gbs[:], wgs[:], preferred_element_type=f32)


def _fold_weights(p):
    f32 = jnp.float32
    a1 = p['ca_kvpi_w'][D:].T
    a2 = p['ca_opt_w'].T
    b1 = p['ca_kvpt_w'][D:].T
    b2 = p['ca_opi_w'].T
    bvi = p['ca_kvpi_b'][D:][None]
    bot = p['ca_opt_b'][None]
    bvt = p['ca_kvpt_b'][D:][None]
    boi = p['ca_opi_b'][None]
    gwc = p['cb_gds_w'][:, 0][None]
    gbc = p['cb_gds_b'][None]
    wgc = p['cb_c0_w'][:, 2 * D:].T
    gws = p['sh_gds_w'][:, 0][None]
    gbs = p['sh_gds_b'][None]
    wgs = p['sh_h0_w'][:, 2 * D:].T
    shapes = [
        jax.ShapeDtypeStruct((D, D), f32),
        jax.ShapeDtypeStruct((D, D), f32),
        jax.ShapeDtypeStruct((1, D), f32),
        jax.ShapeDtypeStruct((1, D), f32),
        jax.ShapeDtypeStruct((1, D), f32),
        jax.ShapeDtypeStruct((1, D), f32),
        jax.ShapeDtypeStruct((1, 128), f32),
        jax.ShapeDtypeStruct((1, 128), f32),
    ]
    return pl.pallas_call(_prep_kernel, out_shape=shapes)(
        a1, a2, b1, b2, bvi, bot, bvt, boi, gwc, gbc, wgc, gws, gbs, wgs)


def _gelu_exact(x):
    return 0.5 * x * (1.0 + jax.lax.erf(x * 0.7071067811865476))


def _ln(x, g, b, eps=1e-5):
    m = jnp.mean(x, axis=-1, keepdims=True)
    c = x - m
    v = jnp.mean(c * c, axis=-1, keepdims=True)
    return c * jax.lax.rsqrt(v + eps) * g + b


def _main_kernel(xt_ref, xi_ref, g_ref,
                 ct_ref, ci_ref, bt_ref, bi_ref,
                 lntg_ref, lntb_ref, lnig_ref, lnib_ref,
                 wt_ref, wi_ref, u_ref, brow_ref,
                 w1c_ref, b1c_ref, n1_ref, b1n_ref, b0n_ref,
                 whead_ref, bhead_ref, lt_ref,
                 out_ref):
    f32 = jnp.float32
    bf16 = jnp.bfloat16
    xt = xt_ref[:]
    xi = xi_ref[:]
    g = g_ref[:]
    pt = jnp.dot(xt.astype(bf16), ct_ref[:], preferred_element_type=f32)
    pi = jnp.dot(xi.astype(bf16), ci_ref[:], preferred_element_type=f32)
    i_out = pt[:, :D] + bi_ref[:]
    n_t = pt[:, D:]
    t_out = pi[:, :D] + bt_ref[:]
    n_i = pi[:, D:]
    t_refv = _ln(xt + t_out, lntg_ref[:], lntb_ref[:])
    i_refv = _ln(xi + i_out, lnig_ref[:], lnib_ref[:])
    q = (jnp.dot(t_refv.astype(bf16), wt_ref[:], preferred_element_type=f32)
         + jnp.dot(i_refv.astype(bf16), wi_ref[:], preferred_element_type=f32)
         + g * u_ref[:] + brow_ref[:])
    qa = _gelu_exact(q)
    h0 = qa[:, :D]
    hs = qa[:, D:]
    h1 = _gelu_exact(
        jnp.dot(h0.astype(bf16), w1c_ref[:], preferred_element_type=f32) + b1c_ref[:])
    n0 = _gelu_exact(n_t + n_i + b0n_ref[:])
    n1 = _gelu_exact(
        jnp.dot(n0.astype(bf16), n1_ref[:], preferred_element_type=f32) + b1n_ref[:])
    hcat = jnp.concatenate([h1, n1, hs], axis=1).astype(bf16)
    heads = jnp.dot(hcat, whead_ref[:], preferred_element_type=f32) + bhead_ref[:]
    conflict = heads[:, 0:3]
    normal = heads[:, 3:6]
    sarcasm = heads[:, 6:8]
    tau = jax.nn.sigmoid(lt_ref[:])
    gate = jax.nn.sigmoid((g - tau) * TEMPERATURE)
    logits = gate * conflict + (1.0 - gate) * normal
    routing = (g > tau).astype(f32)
    pad = jnp.zeros_like(heads[:, 0:4])
    out_ref[:] = jnp.concatenate(
        [logits, normal, conflict, sarcasm, routing, pad], axis=1)


@jax.jit
def _run(s_t, s_i, gds, params):
    f32 = jnp.float32
    p = params
    mt, mi, bt, bi, uc, cc, us, cs = _fold_weights(p)
    bf16 = jnp.bfloat16
    c_t = jnp.concatenate([mi, p['nb_m0_w'][:, :D].T], axis=1).astype(bf16)
    c_i = jnp.concatenate([mt, p['nb_m0_w'][:, D:].T], axis=1).astype(bf16)
    w_t = jnp.concatenate([p['cb_c0_w'][:, :D].T, p['sh_h0_w'][:, :D].T], axis=1).astype(bf16)
    w_i = jnp.concatenate([p['cb_c0_w'][:, D:2 * D].T, p['sh_h0_w'][:, D:2 * D].T], axis=1).astype(bf16)
    u_row = jnp.concatenate([uc, us], axis=1)
    b_row = jnp.concatenate([p['cb_c0_b'][None] + cc, p['sh_h0_b'][None] + cs], axis=1)
    w1c = p['cb_c1_w'].T.astype(bf16)
    b1c = p['cb_c1_b'][None]
    n1w = p['nb_m1_w'].T.astype(bf16)
    b1n = p['nb_m1_b'][None]
    b0n = p['nb_m0_b'][None]
    whead = jnp.zeros((384, OUT_W), f32)
    whead = whead.at[0:128, 0:3].set(p['cb_c2_w'].T)
    whead = whead.at[128:256, 3:6].set(p['nb_m2_w'].T)
    whead = whead.at[256:384, 6:8].set(p['sh_h1_w'].T).astype(bf16)
    bhead = jnp.zeros((1, OUT_W), f32)
    bhead = bhead.at[0, 0:3].set(p['cb_c2_b'])
    bhead = bhead.at[0, 3:6].set(p['nb_m2_b'])
    bhead = bhead.at[0, 6:8].set(p['sh_h1_b'])
    lt = p['log_threshold'].reshape(1, 1)
    gds2 = gds[:, None]

    B = s_t.shape[0]
    grid = (B // BLOCK,)
    row = lambda i: (i, 0)
    rep = lambda i: (0, 0)
    in_specs = [
        pl.BlockSpec((BLOCK, D), row),
        pl.BlockSpec((BLOCK, D), row),
        pl.BlockSpec((BLOCK, 1), row),
        pl.BlockSpec((D, 2 * D), rep),
        pl.BlockSpec((D, 2 * D), rep),
        pl.BlockSpec((1, D), rep),
        pl.BlockSpec((1, D), rep),
        pl.BlockSpec((1, D), rep),
        pl.BlockSpec((1, D), rep),
        pl.BlockSpec((1, D), rep),
        pl.BlockSpec((1, D), rep),
        pl.BlockSpec((D, 384), rep),
        pl.BlockSpec((D, 384), rep),
        pl.BlockSpec((1, 384), rep),
        pl.BlockSpec((1, 384), rep),
        pl.BlockSpec((D, 128), rep),
        pl.BlockSpec((1, 128), rep),
        pl.BlockSpec((D, 128), rep),
        pl.BlockSpec((1, 128), rep),
        pl.BlockSpec((1, D), rep),
        pl.BlockSpec((384, OUT_W), rep),
        pl.BlockSpec((1, OUT_W), rep),
        pl.BlockSpec((1, 1), rep),
    ]
    packed = pl.pallas_call(
        _main_kernel,
        grid=grid,
        in_specs=in_specs,
        out_specs=pl.BlockSpec((BLOCK, OUT_W), row),
        out_shape=jax.ShapeDtypeStruct((B, OUT_W), f32),
        compiler_params=pltpu.CompilerParams(
            dimension_semantics=("arbitrary",)),
    )(s_t, s_i, gds2, c_t, c_i, bt, bi,
      p['ca_lnt_g'][None], p['ca_lnt_b'][None],
      p['ca_lni_g'][None], p['ca_lni_b'][None],
      w_t, w_i, u_row, b_row, w1c, b1c, n1w, b1n, b0n,
      whead, bhead, lt)
    logits = packed[:, 0:3]
    normal = packed[:, 3:6]
    conflict = packed[:, 6:9]
    sarcasm = packed[:, 9:11]
    routing = packed[:, 11]
    return logits, routing, normal, conflict, sarcasm


def kernel(s_t, s_i, gds, params):
    return _run(s_t, s_i, gds, params)


# speedup vs baseline: 1.2370x; 1.2370x over previous
import jax
import jax.numpy as jnp
from jax.experimental import pallas as pl
from jax.experimental.pallas import tpu as pltpu

D = 256
TEMPERATURE = 10.0
BLOCK = 512

_NT = (((1,), (1,)), ((), ()))


def _dnt(a, b):
    return jax.lax.dot_general(a, b, dimension_numbers=_NT,
                               preferred_element_type=jnp.float32)


def _gelu_exact(x):
    return 0.5 * x * (1.0 + jax.lax.erf(x * 0.7071067811865476))


def _ln(x, g, b, eps=1e-5):
    m = jnp.mean(x, axis=-1, keepdims=True)
    c = x - m
    v = jnp.mean(c * c, axis=-1, keepdims=True)
    return c * jax.lax.rsqrt(v + eps) * g + b


def _kernel_body(xt_ref, xi_ref, g_ref,
                 wvi_ref, wot_ref, wvt_ref, woi_ref,
                 bvi_ref, bot_ref, bvt_ref, boi_ref,
                 lntg_ref, lntb_ref, lnig_ref, lnib_ref,
                 c0_ref, c0b_ref, gwc_ref, gbc_ref,
                 sh0_ref, sh0b_ref, gws_ref, gbs_ref,
                 c1_ref, c1b_ref,
                 n0_ref, n0b_ref, n1_ref, n1b_ref,
                 c2_ref, c2b_ref, m2_ref, m2b_ref, sh1_ref, sh1b_ref,
                 lt_ref,
                 logits_ref, routing_ref, normal_ref, conflict_ref,
                 sarcasm_ref,
                 s_ft, s_fi, s_bt, s_bi, s_uc, s_bc, s_us, s_bs):
    f32 = jnp.float32
    bf16 = jnp.bfloat16

    @pl.when(pl.program_id(0) == 0)
    def _fold():
        wot = wot_ref[:].astype(bf16)
        wvi = wvi_ref[:].astype(bf16)
        woi = woi_ref[:].astype(bf16)
        wvt = wvt_ref[:].astype(bf16)
        s_ft[:] = jnp.dot(wot, wvi, preferred_element_type=f32).astype(bf16)
        s_fi[:] = jnp.dot(woi, wvt, preferred_element_type=f32).astype(bf16)
        s_bt[:] = _dnt(bvi_ref[:], wot_ref[:]) + bot_ref[:]
        s_bi[:] = _dnt(bvt_ref[:], woi_ref[:]) + boi_ref[:]
        wgc = c0_ref[:, 2 * D:2 * D + 32]
        wgs = sh0_ref[:, 2 * D:2 * D + 32]
        s_uc[:] = _dnt(gwc_ref[:], wgc)
        s_bc[:] = _dnt(gbc_ref[:], wgc) + c0b_ref[:]
        s_us[:] = _dnt(gws_ref[:], wgs)
        s_bs[:] = _dnt(gbs_ref[:], wgs) + sh0b_ref[:]

    xt = xt_ref[:]
    xi = xi_ref[:]
    g = g_ref[:]
    xtb = xt.astype(bf16)
    xib = xi.astype(bf16)
    t_out = _dnt(xib, s_ft[:]) + s_bt[:]
    i_out = _dnt(xtb, s_fi[:]) + s_bi[:]
    t_refv = _ln(xt + t_out, lntg_ref[:], lntb_ref[:])
    i_refv = _ln(xi + i_out, lnig_ref[:], lnib_ref[:])
    trb = t_refv.astype(bf16)
    irb = i_refv.astype(bf16)
    c0a = c0_ref[:, :D].astype(bf16)
    c0bw = c0_ref[:, D:2 * D].astype(bf16)
    q = _dnt(trb, c0a) + _dnt(irb, c0bw) + g * s_uc[:] + s_bc[:]
    h0 = _gelu_exact(q)
    sh0a = sh0_ref[:, :D].astype(bf16)
    sh0bw = sh0_ref[:, D:2 * D].astype(bf16)
    hsq = _dnt(trb, sh0a) + _dnt(irb, sh0bw) + g * s_us[:] + s_bs[:]
    hs = _gelu_exact(hsq)
    h1 = _gelu_exact(_dnt(h0.astype(bf16), c1_ref[:].astype(bf16)) + c1b_ref[:])
    n_pre = (_dnt(xtb, n0_ref[:, :D].astype(bf16))
             + _dnt(xib, n0_ref[:, D:].astype(bf16)) + n0b_ref[:])
    n0 = _gelu_exact(n_pre)
    n1 = _gelu_exact(_dnt(n0.astype(bf16), n1_ref[:].astype(bf16)) + n1b_ref[:])
    conflict = _dnt(h1.astype(bf16), c2_ref[:].astype(bf16)) + c2b_ref[:]
    normal = _dnt(n1.astype(bf16), m2_ref[:].astype(bf16)) + m2b_ref[:]
    sarcasm = _dnt(hs.astype(bf16), sh1_ref[:].astype(bf16)) + sh1b_ref[:]
    tau = jax.nn.sigmoid(lt_ref[:])
    gate = jax.nn.sigmoid((g - tau) * TEMPERATURE)
    logits_ref[:] = gate * conflict + (1.0 - gate) * normal
    routing_ref[:] = (g > tau).astype(f32)
    normal_ref[:] = normal
    conflict_ref[:] = conflict
    sarcasm_ref[:] = sarcasm


@jax.jit
def _run(s_t, s_i, gds, params):
    f32 = jnp.float32
    bf16 = jnp.bfloat16
    p = params
    B = s_t.shape[0]
    grid = (B // BLOCK,)
    row = lambda i: (i, 0)
    rep = lambda i: (0, 0)
    vhalf = lambda i: (1, 0)

    in_specs = [
        pl.BlockSpec((BLOCK, D), row),
        pl.BlockSpec((BLOCK, D), row),
        pl.BlockSpec((BLOCK, 1), row),
        pl.BlockSpec((D, D), vhalf),
        pl.BlockSpec((D, D), rep),
        pl.BlockSpec((D, D), vhalf),
        pl.BlockSpec((D, D), rep),
        pl.BlockSpec((1, D), rep),
        pl.BlockSpec((1, D), rep),
        pl.BlockSpec((1, D), rep),
        pl.BlockSpec((1, D), rep),
        pl.BlockSpec((1, D), rep),
        pl.BlockSpec((1, D), rep),
        pl.BlockSpec((1, D), rep),
        pl.BlockSpec((1, D), rep),
        pl.BlockSpec((D, 2 * D + 32), rep),
        pl.BlockSpec((1, D), rep),
        pl.BlockSpec((1, 32), rep),
        pl.BlockSpec((1, 32), rep),
        pl.BlockSpec((128, 2 * D + 32), rep),
        pl.BlockSpec((1, 128), rep),
        pl.BlockSpec((1, 32), rep),
        pl.BlockSpec((1, 32), rep),
        pl.BlockSpec((128, D), rep),
        pl.BlockSpec((1, 128), rep),
        pl.BlockSpec((D, 2 * D), rep),
        pl.BlockSpec((1, D), rep),
        pl.BlockSpec((128, D), rep),
        pl.BlockSpec((1, 128), rep),
        pl.BlockSpec((3, 128), rep),
        pl.BlockSpec((1, 3), rep),
        pl.BlockSpec((3, 128), rep),
        pl.BlockSpec((1, 3), rep),
        pl.BlockSpec((2, 128), rep),
        pl.BlockSpec((1, 2), rep),
        pl.BlockSpec((1, 1), rep),
    ]
    out_specs = [
        pl.BlockSpec((BLOCK, 3), row),
        pl.BlockSpec((BLOCK, 1), row),
        pl.BlockSpec((BLOCK, 3), row),
        pl.BlockSpec((BLOCK, 3), row),
        pl.BlockSpec((BLOCK, 2), row),
    ]
    out_shape = [
        jax.ShapeDtypeStruct((B, 3), f32),
        jax.ShapeDtypeStruct((B, 1), f32),
        jax.ShapeDtypeStruct((B, 3), f32),
        jax.ShapeDtypeStruct((B, 3), f32),
        jax.ShapeDtypeStruct((B, 2), f32),
    ]
    scratch_shapes = [
        pltpu.VMEM((D, D), bf16),
        pltpu.VMEM((D, D), bf16),
        pltpu.VMEM((1, D), f32),
        pltpu.VMEM((1, D), f32),
        pltpu.VMEM((1, D), f32),
        pltpu.VMEM((1, D), f32),
        pltpu.VMEM((1, 128), f32),
        pltpu.VMEM((1, 128), f32),
    ]
    outs = pl.pallas_call(
        _kernel_body,
        grid=grid,
        in_specs=in_specs,
        out_specs=out_specs,
        out_shape=out_shape,
        scratch_shapes=scratch_shapes,
        compiler_params=pltpu.CompilerParams(
            dimension_semantics=("arbitrary",)),
    )(s_t, s_i, gds[:, None],
      p['ca_kvpi_w'], p['ca_opt_w'], p['ca_kvpt_w'], p['ca_opi_w'],
      p['ca_kvpi_b'][D:][None], p['ca_opt_b'][None],
      p['ca_kvpt_b'][D:][None], p['ca_opi_b'][None],
      p['ca_lnt_g'][None], p['ca_lnt_b'][None],
      p['ca_lni_g'][None], p['ca_lni_b'][None],
      p['cb_c0_w'], p['cb_c0_b'][None],
      p['cb_gds_w'][:, 0][None], p['cb_gds_b'][None],
      p['sh_h0_w'], p['sh_h0_b'][None],
      p['sh_gds_w'][:, 0][None], p['sh_gds_b'][None],
      p['cb_c1_w'], p['cb_c1_b'][None],
      p['nb_m0_w'], p['nb_m0_b'][None],
      p['nb_m1_w'], p['nb_m1_b'][None],
      p['cb_c2_w'], p['cb_c2_b'][None],
      p['nb_m2_w'], p['nb_m2_b'][None],
      p['sh_h1_w'], p['sh_h1_b'][None],
      p['log_threshold'].reshape(1, 1))
    logits, routing, normal, conflict, sarcasm = outs
    return logits, routing[:, 0], normal, conflict, sarcasm


def kernel(s_t, s_i, gds, params):
    return _run(s_t, s_i, gds, params)


# speedup vs baseline: 1.4383x; 1.1627x over previous
import jax
import jax.numpy as jnp
from jax.experimental import pallas as pl
from jax.experimental.pallas import tpu as pltpu

D = 256
TEMPERATURE = 10.0
BLOCK = 1024

_NT = (((1,), (1,)), ((), ()))


def _dnt(a, b):
    return jax.lax.dot_general(a, b, dimension_numbers=_NT,
                               preferred_element_type=jnp.float32)


def _gelu_exact(x):
    return 0.5 * x * (1.0 + jax.lax.erf(x * 0.7071067811865476))


def _ln(x, g, b, eps=1e-5):
    m = jnp.mean(x, axis=-1, keepdims=True)
    c = x - m
    v = jnp.mean(c * c, axis=-1, keepdims=True)
    return c * jax.lax.rsqrt(v + eps) * g + b


def _kernel_body(xt_ref, xi_ref, g_ref,
                 wvi_ref, wot_ref, wvt_ref, woi_ref,
                 bvi_ref, bot_ref, bvt_ref, boi_ref,
                 lntg_ref, lntb_ref, lnig_ref, lnib_ref,
                 c0_ref, c0b_ref, gwc_ref, gbc_ref,
                 sh0_ref, sh0b_ref, gws_ref, gbs_ref,
                 c1_ref, c1b_ref,
                 n0_ref, n0b_ref, n1_ref, n1b_ref,
                 c2_ref, c2b_ref, m2_ref, m2b_ref, sh1_ref, sh1b_ref,
                 lt_ref,
                 logits_ref, routing_ref, normal_ref, conflict_ref,
                 sarcasm_ref,
                 s_ft, s_fi, s_bt, s_bi, s_uc, s_bc, s_us, s_bs,
                 s_c0a, s_c0b, s_sh0a, s_sh0b, s_c1, s_n0a, s_n0b, s_n1):
    f32 = jnp.float32
    bf16 = jnp.bfloat16

    @pl.when(pl.program_id(0) == 0)
    def _fold():
        wot = wot_ref[:].astype(bf16)
        wvi = wvi_ref[:].astype(bf16)
        woi = woi_ref[:].astype(bf16)
        wvt = wvt_ref[:].astype(bf16)
        s_ft[:] = jnp.dot(wot, wvi, preferred_element_type=f32).astype(bf16)
        s_fi[:] = jnp.dot(woi, wvt, preferred_element_type=f32).astype(bf16)
        s_bt[:] = _dnt(bvi_ref[:], wot_ref[:]) + bot_ref[:]
        s_bi[:] = _dnt(bvt_ref[:], woi_ref[:]) + boi_ref[:]
        wgc = c0_ref[:, 2 * D:2 * D + 32]
        wgs = sh0_ref[:, 2 * D:2 * D + 32]
        s_uc[:] = _dnt(gwc_ref[:], wgc)
        s_bc[:] = _dnt(gbc_ref[:], wgc) + c0b_ref[:]
        s_us[:] = _dnt(gws_ref[:], wgs)
        s_bs[:] = _dnt(gbs_ref[:], wgs) + sh0b_ref[:]
        s_c0a[:] = c0_ref[:, :D].astype(bf16)
        s_c0b[:] = c0_ref[:, D:2 * D].astype(bf16)
        s_sh0a[:] = sh0_ref[:, :D].astype(bf16)
        s_sh0b[:] = sh0_ref[:, D:2 * D].astype(bf16)
        s_c1[:] = c1_ref[:].astype(bf16)
        s_n0a[:] = n0_ref[:, :D].astype(bf16)
        s_n0b[:] = n0_ref[:, D:].astype(bf16)
        s_n1[:] = n1_ref[:].astype(bf16)

    xt = xt_ref[:]
    xi = xi_ref[:]
    g = g_ref[:]
    xtb = xt.astype(bf16)
    xib = xi.astype(bf16)
    t_out = _dnt(xib, s_ft[:]) + s_bt[:]
    i_out = _dnt(xtb, s_fi[:]) + s_bi[:]
    t_refv = _ln(xt + t_out, lntg_ref[:], lntb_ref[:])
    i_refv = _ln(xi + i_out, lnig_ref[:], lnib_ref[:])
    trb = t_refv.astype(bf16)
    irb = i_refv.astype(bf16)
    q = _dnt(trb, s_c0a[:]) + _dnt(irb, s_c0b[:]) + g * s_uc[:] + s_bc[:]
    h0 = _gelu_exact(q)
    hsq = _dnt(trb, s_sh0a[:]) + _dnt(irb, s_sh0b[:]) + g * s_us[:] + s_bs[:]
    hs = _gelu_exact(hsq)
    h1 = _gelu_exact(_dnt(h0.astype(bf16), s_c1[:]) + c1b_ref[:])
    n_pre = _dnt(xtb, s_n0a[:]) + _dnt(xib, s_n0b[:]) + n0b_ref[:]
    n0 = _gelu_exact(n_pre)
    n1 = _gelu_exact(_dnt(n0.astype(bf16), s_n1[:]) + n1b_ref[:])
    conflict = _dnt(h1.astype(bf16), c2_ref[:].astype(bf16)) + c2b_ref[:]
    normal = _dnt(n1.astype(bf16), m2_ref[:].astype(bf16)) + m2b_ref[:]
    sarcasm = _dnt(hs.astype(bf16), sh1_ref[:].astype(bf16)) + sh1b_ref[:]
    tau = jax.nn.sigmoid(lt_ref[:])
    gate = jax.nn.sigmoid((g - tau) * TEMPERATURE)
    logits_ref[:] = gate * conflict + (1.0 - gate) * normal
    routing_ref[:] = (g > tau).astype(f32)
    normal_ref[:] = normal
    conflict_ref[:] = conflict
    sarcasm_ref[:] = sarcasm


@jax.jit
def _run(s_t, s_i, gds, params):
    f32 = jnp.float32
    bf16 = jnp.bfloat16
    p = params
    B = s_t.shape[0]
    grid = (B // BLOCK,)
    row = lambda i: (i, 0)
    rep = lambda i: (0, 0)
    vhalf = lambda i: (1, 0)

    in_specs = [
        pl.BlockSpec((BLOCK, D), row),
        pl.BlockSpec((BLOCK, D), row),
        pl.BlockSpec((BLOCK, 1), row),
        pl.BlockSpec((D, D), vhalf),
        pl.BlockSpec((D, D), rep),
        pl.BlockSpec((D, D), vhalf),
        pl.BlockSpec((D, D), rep),
        pl.BlockSpec((1, D), rep),
        pl.BlockSpec((1, D), rep),
        pl.BlockSpec((1, D), rep),
        pl.BlockSpec((1, D), rep),
        pl.BlockSpec((1, D), rep),
        pl.BlockSpec((1, D), rep),
        pl.BlockSpec((1, D), rep),
        pl.BlockSpec((1, D), rep),
        pl.BlockSpec((D, 2 * D + 32), rep),
        pl.BlockSpec((1, D), rep),
        pl.BlockSpec((1, 32), rep),
        pl.BlockSpec((1, 32), rep),
        pl.BlockSpec((128, 2 * D + 32), rep),
        pl.BlockSpec((1, 128), rep),
        pl.BlockSpec((1, 32), rep),
        pl.BlockSpec((1, 32), rep),
        pl.BlockSpec((128, D), rep),
        pl.BlockSpec((1, 128), rep),
        pl.BlockSpec((D, 2 * D), rep),
        pl.BlockSpec((1, D), rep),
        pl.BlockSpec((128, D), rep),
        pl.BlockSpec((1, 128), rep),
        pl.BlockSpec((3, 128), rep),
        pl.BlockSpec((1, 3), rep),
        pl.BlockSpec((3, 128), rep),
        pl.BlockSpec((1, 3), rep),
        pl.BlockSpec((2, 128), rep),
        pl.BlockSpec((1, 2), rep),
        pl.BlockSpec((1, 1), rep),
    ]
    out_specs = [
        pl.BlockSpec((BLOCK, 3), row),
        pl.BlockSpec((BLOCK, 1), row),
        pl.BlockSpec((BLOCK, 3), row),
        pl.BlockSpec((BLOCK, 3), row),
        pl.BlockSpec((BLOCK, 2), row),
    ]
    out_shape = [
        jax.ShapeDtypeStruct((B, 3), f32),
        jax.ShapeDtypeStruct((B, 1), f32),
        jax.ShapeDtypeStruct((B, 3), f32),
        jax.ShapeDtypeStruct((B, 3), f32),
        jax.ShapeDtypeStruct((B, 2), f32),
    ]
    scratch_shapes = [
        pltpu.VMEM((D, D), bf16),
        pltpu.VMEM((D, D), bf16),
        pltpu.VMEM((1, D), f32),
        pltpu.VMEM((1, D), f32),
        pltpu.VMEM((1, D), f32),
        pltpu.VMEM((1, D), f32),
        pltpu.VMEM((1, 128), f32),
        pltpu.VMEM((1, 128), f32),
        pltpu.VMEM((D, D), bf16),
        pltpu.VMEM((D, D), bf16),
        pltpu.VMEM((128, D), bf16),
        pltpu.VMEM((128, D), bf16),
        pltpu.VMEM((128, D), bf16),
        pltpu.VMEM((D, D), bf16),
        pltpu.VMEM((D, D), bf16),
        pltpu.VMEM((128, D), bf16),
    ]
    outs = pl.pallas_call(
        _kernel_body,
        grid=grid,
        in_specs=in_specs,
        out_specs=out_specs,
        out_shape=out_shape,
        scratch_shapes=scratch_shapes,
        compiler_params=pltpu.CompilerParams(
            dimension_semantics=("arbitrary",)),
    )(s_t, s_i, gds[:, None],
      p['ca_kvpi_w'], p['ca_opt_w'], p['ca_kvpt_w'], p['ca_opi_w'],
      p['ca_kvpi_b'][D:][None], p['ca_opt_b'][None],
      p['ca_kvpt_b'][D:][None], p['ca_opi_b'][None],
      p['ca_lnt_g'][None], p['ca_lnt_b'][None],
      p['ca_lni_g'][None], p['ca_lni_b'][None],
      p['cb_c0_w'], p['cb_c0_b'][None],
      p['cb_gds_w'][:, 0][None], p['cb_gds_b'][None],
      p['sh_h0_w'], p['sh_h0_b'][None],
      p['sh_gds_w'][:, 0][None], p['sh_gds_b'][None],
      p['cb_c1_w'], p['cb_c1_b'][None],
      p['nb_m0_w'], p['nb_m0_b'][None],
      p['nb_m1_w'], p['nb_m1_b'][None],
      p['cb_c2_w'], p['cb_c2_b'][None],
      p['nb_m2_w'], p['nb_m2_b'][None],
      p['sh_h1_w'], p['sh_h1_b'][None],
      p['log_threshold'].reshape(1, 1))
    logits, routing, normal, conflict, sarcasm = outs
    return logits, routing[:, 0], normal, conflict, sarcasm


def kernel(s_t, s_i, gds, params):
    return _run(s_t, s_i, gds, params)


# speedup vs baseline: 1.5484x; 1.0766x over previous
import jax
import jax.numpy as jnp
from jax.experimental import pallas as pl
from jax.experimental.pallas import tpu as pltpu

D = 256
TEMPERATURE = 10.0
BLOCK = 2048

_NT = (((1,), (1,)), ((), ()))


def _dnt(a, b):
    return jax.lax.dot_general(a, b, dimension_numbers=_NT,
                               preferred_element_type=jnp.float32)


def _gelu_exact(x):
    return 0.5 * x * (1.0 + jax.lax.erf(x * 0.7071067811865476))


def _ln(x, g, b, eps=1e-5):
    m = jnp.mean(x, axis=-1, keepdims=True)
    c = x - m
    v = jnp.mean(c * c, axis=-1, keepdims=True)
    return c * jax.lax.rsqrt(v + eps) * g + b


def _kernel_body(xt_ref, xi_ref, g_ref,
                 wvi_ref, wot_ref, wvt_ref, woi_ref,
                 bvi_ref, bot_ref, bvt_ref, boi_ref,
                 lntg_ref, lntb_ref, lnig_ref, lnib_ref,
                 c0_ref, c0b_ref, gwc_ref, gbc_ref,
                 sh0_ref, sh0b_ref, gws_ref, gbs_ref,
                 c1_ref, c1b_ref,
                 n0_ref, n0b_ref, n1_ref, n1b_ref,
                 c2_ref, c2b_ref, m2_ref, m2b_ref, sh1_ref, sh1b_ref,
                 lt_ref,
                 logits_ref, routing_ref, normal_ref, conflict_ref,
                 sarcasm_ref,
                 s_ft, s_fi, s_bt, s_bi, s_uc, s_bc, s_us, s_bs,
                 s_c0a, s_c0b, s_sh0a, s_sh0b, s_c1, s_n0a, s_n0b, s_n1):
    f32 = jnp.float32
    bf16 = jnp.bfloat16

    @pl.when(pl.program_id(0) == 0)
    def _fold():
        wot = wot_ref[:].astype(bf16)
        wvi = wvi_ref[:].astype(bf16)
        woi = woi_ref[:].astype(bf16)
        wvt = wvt_ref[:].astype(bf16)
        s_ft[:] = jnp.dot(wot, wvi, preferred_element_type=f32).astype(bf16)
        s_fi[:] = jnp.dot(woi, wvt, preferred_element_type=f32).astype(bf16)
        s_bt[:] = _dnt(bvi_ref[:], wot_ref[:]) + bot_ref[:]
        s_bi[:] = _dnt(bvt_ref[:], woi_ref[:]) + boi_ref[:]
        wgc = c0_ref[:, 2 * D:2 * D + 32]
        wgs = sh0_ref[:, 2 * D:2 * D + 32]
        s_uc[:] = _dnt(gwc_ref[:], wgc)
        s_bc[:] = _dnt(gbc_ref[:], wgc) + c0b_ref[:]
        s_us[:] = _dnt(gws_ref[:], wgs)
        s_bs[:] = _dnt(gbs_ref[:], wgs) + sh0b_ref[:]
        s_c0a[:] = c0_ref[:, :D].astype(bf16)
        s_c0b[:] = c0_ref[:, D:2 * D].astype(bf16)
        s_sh0a[:] = sh0_ref[:, :D].astype(bf16)
        s_sh0b[:] = sh0_ref[:, D:2 * D].astype(bf16)
        s_c1[:] = c1_ref[:].astype(bf16)
        s_n0a[:] = n0_ref[:, :D].astype(bf16)
        s_n0b[:] = n0_ref[:, D:].astype(bf16)
        s_n1[:] = n1_ref[:].astype(bf16)

    xt = xt_ref[:]
    xi = xi_ref[:]
    g = g_ref[:]
    xtb = xt.astype(bf16)
    xib = xi.astype(bf16)
    t_out = _dnt(xib, s_ft[:]) + s_bt[:]
    i_out = _dnt(xtb, s_fi[:]) + s_bi[:]
    t_refv = _ln(xt + t_out, lntg_ref[:], lntb_ref[:])
    i_refv = _ln(xi + i_out, lnig_ref[:], lnib_ref[:])
    trb = t_refv.astype(bf16)
    irb = i_refv.astype(bf16)
    q = _dnt(trb, s_c0a[:]) + _dnt(irb, s_c0b[:]) + g * s_uc[:] + s_bc[:]
    h0 = _gelu_exact(q)
    hsq = _dnt(trb, s_sh0a[:]) + _dnt(irb, s_sh0b[:]) + g * s_us[:] + s_bs[:]
    hs = _gelu_exact(hsq)
    h1 = _gelu_exact(_dnt(h0.astype(bf16), s_c1[:]) + c1b_ref[:])
    n_pre = _dnt(xtb, s_n0a[:]) + _dnt(xib, s_n0b[:]) + n0b_ref[:]
    n0 = _gelu_exact(n_pre)
    n1 = _gelu_exact(_dnt(n0.astype(bf16), s_n1[:]) + n1b_ref[:])
    conflict = _dnt(h1.astype(bf16), c2_ref[:].astype(bf16)) + c2b_ref[:]
    normal = _dnt(n1.astype(bf16), m2_ref[:].astype(bf16)) + m2b_ref[:]
    sarcasm = _dnt(hs.astype(bf16), sh1_ref[:].astype(bf16)) + sh1b_ref[:]
    tau = jax.nn.sigmoid(lt_ref[:])
    gate = jax.nn.sigmoid((g - tau) * TEMPERATURE)
    logits_ref[:] = gate * conflict + (1.0 - gate) * normal
    routing_ref[:] = (g > tau).astype(f32)
    normal_ref[:] = normal
    conflict_ref[:] = conflict
    sarcasm_ref[:] = sarcasm


@jax.jit
def _run(s_t, s_i, gds, params):
    f32 = jnp.float32
    bf16 = jnp.bfloat16
    p = params
    B = s_t.shape[0]
    grid = (B // BLOCK,)
    row = lambda i: (i, 0)
    rep = lambda i: (0, 0)
    vhalf = lambda i: (1, 0)

    in_specs = [
        pl.BlockSpec((BLOCK, D), row),
        pl.BlockSpec((BLOCK, D), row),
        pl.BlockSpec((BLOCK, 1), row),
        pl.BlockSpec((D, D), vhalf),
        pl.BlockSpec((D, D), rep),
        pl.BlockSpec((D, D), vhalf),
        pl.BlockSpec((D, D), rep),
        pl.BlockSpec((1, D), rep),
        pl.BlockSpec((1, D), rep),
        pl.BlockSpec((1, D), rep),
        pl.BlockSpec((1, D), rep),
        pl.BlockSpec((1, D), rep),
        pl.BlockSpec((1, D), rep),
        pl.BlockSpec((1, D), rep),
        pl.BlockSpec((1, D), rep),
        pl.BlockSpec((D, 2 * D + 32), rep),
        pl.BlockSpec((1, D), rep),
        pl.BlockSpec((1, 32), rep),
        pl.BlockSpec((1, 32), rep),
        pl.BlockSpec((128, 2 * D + 32), rep),
        pl.BlockSpec((1, 128), rep),
        pl.BlockSpec((1, 32), rep),
        pl.BlockSpec((1, 32), rep),
        pl.BlockSpec((128, D), rep),
        pl.BlockSpec((1, 128), rep),
        pl.BlockSpec((D, 2 * D), rep),
        pl.BlockSpec((1, D), rep),
        pl.BlockSpec((128, D), rep),
        pl.BlockSpec((1, 128), rep),
        pl.BlockSpec((3, 128), rep),
        pl.BlockSpec((1, 3), rep),
        pl.BlockSpec((3, 128), rep),
        pl.BlockSpec((1, 3), rep),
        pl.BlockSpec((2, 128), rep),
        pl.BlockSpec((1, 2), rep),
        pl.BlockSpec((1, 1), rep),
    ]
    out_specs = [
        pl.BlockSpec((BLOCK, 3), row),
        pl.BlockSpec((BLOCK, 1), row),
        pl.BlockSpec((BLOCK, 3), row),
        pl.BlockSpec((BLOCK, 3), row),
        pl.BlockSpec((BLOCK, 2), row),
    ]
    out_shape = [
        jax.ShapeDtypeStruct((B, 3), f32),
        jax.ShapeDtypeStruct((B, 1), f32),
        jax.ShapeDtypeStruct((B, 3), f32),
        jax.ShapeDtypeStruct((B, 3), f32),
        jax.ShapeDtypeStruct((B, 2), f32),
    ]
    scratch_shapes = [
        pltpu.VMEM((D, D), bf16),
        pltpu.VMEM((D, D), bf16),
        pltpu.VMEM((1, D), f32),
        pltpu.VMEM((1, D), f32),
        pltpu.VMEM((1, D), f32),
        pltpu.VMEM((1, D), f32),
        pltpu.VMEM((1, 128), f32),
        pltpu.VMEM((1, 128), f32),
        pltpu.VMEM((D, D), bf16),
        pltpu.VMEM((D, D), bf16),
        pltpu.VMEM((128, D), bf16),
        pltpu.VMEM((128, D), bf16),
        pltpu.VMEM((128, D), bf16),
        pltpu.VMEM((D, D), bf16),
        pltpu.VMEM((D, D), bf16),
        pltpu.VMEM((128, D), bf16),
    ]
    outs = pl.pallas_call(
        _kernel_body,
        grid=grid,
        in_specs=in_specs,
        out_specs=out_specs,
        out_shape=out_shape,
        scratch_shapes=scratch_shapes,
        compiler_params=pltpu.CompilerParams(
            dimension_semantics=("arbitrary",)),
    )(s_t, s_i, gds[:, None],
      p['ca_kvpi_w'], p['ca_opt_w'], p['ca_kvpt_w'], p['ca_opi_w'],
      p['ca_kvpi_b'][D:][None], p['ca_opt_b'][None],
      p['ca_kvpt_b'][D:][None], p['ca_opi_b'][None],
      p['ca_lnt_g'][None], p['ca_lnt_b'][None],
      p['ca_lni_g'][None], p['ca_lni_b'][None],
      p['cb_c0_w'], p['cb_c0_b'][None],
      p['cb_gds_w'][:, 0][None], p['cb_gds_b'][None],
      p['sh_h0_w'], p['sh_h0_b'][None],
      p['sh_gds_w'][:, 0][None], p['sh_gds_b'][None],
      p['cb_c1_w'], p['cb_c1_b'][None],
      p['nb_m0_w'], p['nb_m0_b'][None],
      p['nb_m1_w'], p['nb_m1_b'][None],
      p['cb_c2_w'], p['cb_c2_b'][None],
      p['nb_m2_w'], p['nb_m2_b'][None],
      p['sh_h1_w'], p['sh_h1_b'][None],
      p['log_threshold'].reshape(1, 1))
    logits, routing, normal, conflict, sarcasm = outs
    return logits, routing[:, 0], normal, conflict, sarcasm


def kernel(s_t, s_i, gds, params):
    return _run(s_t, s_i, gds, params)


# speedup vs baseline: 1.7504x; 1.1304x over previous
import jax
import jax.numpy as jnp
from jax.experimental import pallas as pl
from jax.experimental.pallas import tpu as pltpu

D = 256
TEMPERATURE = 10.0
BLOCK = 2048

_NT = (((1,), (1,)), ((), ()))


def _dnt(a, b):
    return jax.lax.dot_general(a, b, dimension_numbers=_NT,
                               preferred_element_type=jnp.float32)


def _gelu_exact(x):
    return 0.5 * x * (1.0 + jax.lax.erf(x * 0.7071067811865476))


def _ln(x, g, b, eps=1e-5):
    m = jnp.mean(x, axis=-1, keepdims=True)
    c = x - m
    v = jnp.mean(c * c, axis=-1, keepdims=True)
    return c * jax.lax.rsqrt(v + eps) * g + b


def _kernel_body(xt_ref, xi_ref, g_ref,
                 wvi_ref, wot_ref, wvt_ref, woi_ref,
                 bvi_ref, bot_ref, bvt_ref, boi_ref,
                 lntg_ref, lntb_ref, lnig_ref, lnib_ref,
                 c0_ref, c0b_ref, gwc_ref, gbc_ref,
                 sh0_ref, sh0b_ref, gws_ref, gbs_ref,
                 c1_ref, c1b_ref,
                 n0_ref, n0b_ref, n1_ref, n1b_ref,
                 c2_ref, c2b_ref, m2_ref, m2b_ref, sh1_ref, sh1b_ref,
                 lt_ref,
                 logits_ref, routing_ref, normal_ref, conflict_ref,
                 sarcasm_ref,
                 s_xtw, s_xiw, s_bt, s_bi, s_ua, s_ba,
                 s_wt, s_wi, s_c1, s_n1):
    f32 = jnp.float32
    bf16 = jnp.bfloat16

    @pl.when(pl.program_id(0) == 0)
    def _fold():
        wot = wot_ref[:].astype(bf16)
        wvi = wvi_ref[:].astype(bf16)
        woi = woi_ref[:].astype(bf16)
        wvt = wvt_ref[:].astype(bf16)
        s_xiw[0:D, :] = jnp.dot(wot, wvi, preferred_element_type=f32).astype(bf16)
        s_xiw[D:, :] = n0_ref[:, D:].astype(bf16)
        s_xtw[0:D, :] = jnp.dot(woi, wvt, preferred_element_type=f32).astype(bf16)
        s_xtw[D:, :] = n0_ref[:, :D].astype(bf16)
        s_bt[:] = _dnt(bvi_ref[:], wot_ref[:]) + bot_ref[:]
        s_bi[:] = _dnt(bvt_ref[:], woi_ref[:]) + boi_ref[:]
        wgc = c0_ref[:, 2 * D:2 * D + 32]
        wgs = sh0_ref[:, 2 * D:2 * D + 32]
        s_ua[:, 0:D] = _dnt(gwc_ref[:], wgc)
        s_ua[:, D:] = _dnt(gws_ref[:], wgs)
        s_ba[:, 0:D] = _dnt(gbc_ref[:], wgc) + c0b_ref[:]
        s_ba[:, D:] = _dnt(gbs_ref[:], wgs) + sh0b_ref[:]
        s_wt[0:D, :] = c0_ref[:, :D].astype(bf16)
        s_wt[D:, :] = sh0_ref[:, :D].astype(bf16)
        s_wi[0:D, :] = c0_ref[:, D:2 * D].astype(bf16)
        s_wi[D:, :] = sh0_ref[:, D:2 * D].astype(bf16)
        s_c1[:] = c1_ref[:].astype(bf16)
        s_n1[:] = n1_ref[:].astype(bf16)

    xt = xt_ref[:]
    xi = xi_ref[:]
    g = g_ref[:]
    xtb = xt.astype(bf16)
    xib = xi.astype(bf16)
    p_t = _dnt(xtb, s_xtw[:])
    p_i = _dnt(xib, s_xiw[:])
    t_refv = _ln(xt + (p_i[:, :D] + s_bt[:]), lntg_ref[:], lntb_ref[:])
    i_refv = _ln(xi + (p_t[:, :D] + s_bi[:]), lnig_ref[:], lnib_ref[:])
    trb = t_refv.astype(bf16)
    irb = i_refv.astype(bf16)
    q = _dnt(trb, s_wt[:]) + _dnt(irb, s_wi[:]) + g * s_ua[:] + s_ba[:]
    qa = _gelu_exact(q.astype(bf16))
    h0b = qa[:, :D]
    hsb = qa[:, D:]
    h1b = _gelu_exact((_dnt(h0b, s_c1[:]) + c1b_ref[:]).astype(bf16))
    n_pre = p_t[:, D:] + p_i[:, D:] + n0b_ref[:]
    n0b_ = _gelu_exact(n_pre.astype(bf16))
    n1b_ = _gelu_exact((_dnt(n0b_, s_n1[:]) + n1b_ref[:]).astype(bf16))
    conflict = _dnt(h1b, c2_ref[:].astype(bf16)) + c2b_ref[:]
    normal = _dnt(n1b_, m2_ref[:].astype(bf16)) + m2b_ref[:]
    sarcasm = _dnt(hsb, sh1_ref[:].astype(bf16)) + sh1b_ref[:]
    tau = jax.nn.sigmoid(lt_ref[:])
    gate = jax.nn.sigmoid((g - tau) * TEMPERATURE)
    logits_ref[:] = gate * conflict + (1.0 - gate) * normal
    routing_ref[:] = (g > tau).astype(f32)
    normal_ref[:] = normal
    conflict_ref[:] = conflict
    sarcasm_ref[:] = sarcasm


@jax.jit
def _run(s_t, s_i, gds, params):
    f32 = jnp.float32
    bf16 = jnp.bfloat16
    p = params
    B = s_t.shape[0]
    grid = (B // BLOCK,)
    row = lambda i: (i, 0)
    rep = lambda i: (0, 0)
    vhalf = lambda i: (1, 0)

    in_specs = [
        pl.BlockSpec((BLOCK, D), row),
        pl.BlockSpec((BLOCK, D), row),
        pl.BlockSpec((BLOCK, 1), row),
        pl.BlockSpec((D, D), vhalf),
        pl.BlockSpec((D, D), rep),
        pl.BlockSpec((D, D), vhalf),
        pl.BlockSpec((D, D), rep),
        pl.BlockSpec((1, D), rep),
        pl.BlockSpec((1, D), rep),
        pl.BlockSpec((1, D), rep),
        pl.BlockSpec((1, D), rep),
        pl.BlockSpec((1, D), rep),
        pl.BlockSpec((1, D), rep),
        pl.BlockSpec((1, D), rep),
        pl.BlockSpec((1, D), rep),
        pl.BlockSpec((D, 2 * D + 32), rep),
        pl.BlockSpec((1, D), rep),
        pl.BlockSpec((1, 32), rep),
        pl.BlockSpec((1, 32), rep),
        pl.BlockSpec((128, 2 * D + 32), rep),
        pl.BlockSpec((1, 128), rep),
        pl.BlockSpec((1, 32), rep),
        pl.BlockSpec((1, 32), rep),
        pl.BlockSpec((128, D), rep),
        pl.BlockSpec((1, 128), rep),
        pl.BlockSpec((D, 2 * D), rep),
        pl.BlockSpec((1, D), rep),
        pl.BlockSpec((128, D), rep),
        pl.BlockSpec((1, 128), rep),
        pl.BlockSpec((3, 128), rep),
        pl.BlockSpec((1, 3), rep),
        pl.BlockSpec((3, 128), rep),
        pl.BlockSpec((1, 3), rep),
        pl.BlockSpec((2, 128), rep),
        pl.BlockSpec((1, 2), rep),
        pl.BlockSpec((1, 1), rep),
    ]
    out_specs = [
        pl.BlockSpec((BLOCK, 3), row),
        pl.BlockSpec((BLOCK, 1), row),
        pl.BlockSpec((BLOCK, 3), row),
        pl.BlockSpec((BLOCK, 3), row),
        pl.BlockSpec((BLOCK, 2), row),
    ]
    out_shape = [
        jax.ShapeDtypeStruct((B, 3), f32),
        jax.ShapeDtypeStruct((B, 1), f32),
        jax.ShapeDtypeStruct((B, 3), f32),
        jax.ShapeDtypeStruct((B, 3), f32),
        jax.ShapeDtypeStruct((B, 2), f32),
    ]
    scratch_shapes = [
        pltpu.VMEM((2 * D, D), bf16),
        pltpu.VMEM((2 * D, D), bf16),
        pltpu.VMEM((1, D), f32),
        pltpu.VMEM((1, D), f32),
        pltpu.VMEM((1, 384), f32),
        pltpu.VMEM((1, 384), f32),
        pltpu.VMEM((384, D), bf16),
        pltpu.VMEM((384, D), bf16),
        pltpu.VMEM((128, D), bf16),
        pltpu.VMEM((128, D), bf16),
    ]
    outs = pl.pallas_call(
        _kernel_body,
        grid=grid,
        in_specs=in_specs,
        out_specs=out_specs,
        out_shape=out_shape,
        scratch_shapes=scratch_shapes,
        compiler_params=pltpu.CompilerParams(
            dimension_semantics=("arbitrary",)),
    )(s_t, s_i, gds[:, None],
      p['ca_kvpi_w'], p['ca_opt_w'], p['ca_kvpt_w'], p['ca_opi_w'],
      p['ca_kvpi_b'][D:][None], p['ca_opt_b'][None],
      p['ca_kvpt_b'][D:][None], p['ca_opi_b'][None],
      p['ca_lnt_g'][None], p['ca_lnt_b'][None],
      p['ca_lni_g'][None], p['ca_lni_b'][None],
      p['cb_c0_w'], p['cb_c0_b'][None],
      p['cb_gds_w'][:, 0][None], p['cb_gds_b'][None],
      p['sh_h0_w'], p['sh_h0_b'][None],
      p['sh_gds_w'][:, 0][None], p['sh_gds_b'][None],
      p['cb_c1_w'], p['cb_c1_b'][None],
      p['nb_m0_w'], p['nb_m0_b'][None],
      p['nb_m1_w'], p['nb_m1_b'][None],
      p['cb_c2_w'], p['cb_c2_b'][None],
      p['nb_m2_w'], p['nb_m2_b'][None],
      p['sh_h1_w'], p['sh_h1_b'][None],
      p['log_threshold'].reshape(1, 1))
    logits, routing, normal, conflict, sarcasm = outs
    return logits, routing[:, 0], normal, conflict, sarcasm


def kernel(s_t, s_i, gds, params):
    return _run(s_t, s_i, gds, params)


# speedup vs baseline: 1.9260x; 1.1003x over previous
import jax
import jax.numpy as jnp
from jax.experimental import pallas as pl
from jax.experimental.pallas import tpu as pltpu

D = 256
TEMPERATURE = 10.0
BLOCK = 2048

_NT = (((1,), (1,)), ((), ()))


def _dnt(a, b):
    return jax.lax.dot_general(a, b, dimension_numbers=_NT,
                               preferred_element_type=jnp.float32)


def _gelu_exact(x):
    return 0.5 * x * (1.0 + jax.lax.erf(x * 0.7071067811865476))


def _ln(x, g, b, eps=1e-5):
    m = jnp.mean(x, axis=-1, keepdims=True)
    c = x - m
    v = jnp.mean(c * c, axis=-1, keepdims=True)
    return c * jax.lax.rsqrt(v + eps) * g + b


def _kernel_body(xt_ref, xi_ref, g_ref,
                 wvi_ref, wot_ref, wvt_ref, woi_ref,
                 bvi_ref, bot_ref, bvt_ref, boi_ref,
                 lntg_ref, lntb_ref, lnig_ref, lnib_ref,
                 c0_ref, c0b_ref, gwc_ref, gbc_ref,
                 sh0_ref, sh0b_ref, gws_ref, gbs_ref,
                 c1_ref, c1b_ref,
                 n0_ref, n0b_ref, n1_ref, n1b_ref,
                 c2_ref, c2b_ref, m2_ref, m2b_ref, sh1_ref, sh1b_ref,
                 lt_ref,
                 logits_ref, routing_ref, normal_ref, conflict_ref,
                 sarcasm_ref,
                 s_xtw, s_xiw, s_bt, s_bi, s_ua, s_ba,
                 s_wt, s_wi, s_c1, s_n1):
    f32 = jnp.float32
    bf16 = jnp.bfloat16

    @pl.when(pl.program_id(0) == 0)
    def _fold():
        wot = wot_ref[:].astype(bf16)
        wvi = wvi_ref[:].astype(bf16)
        woi = woi_ref[:].astype(bf16)
        wvt = wvt_ref[:].astype(bf16)
        s_xiw[0:D, :] = jnp.dot(wot, wvi, preferred_element_type=f32).astype(bf16)
        s_xiw[D:, :] = n0_ref[:, D:].astype(bf16)
        s_xtw[0:D, :] = jnp.dot(woi, wvt, preferred_element_type=f32).astype(bf16)
        s_xtw[D:, :] = n0_ref[:, :D].astype(bf16)
        s_bt[:] = _dnt(bvi_ref[:], wot_ref[:]) + bot_ref[:]
        s_bi[:] = _dnt(bvt_ref[:], woi_ref[:]) + boi_ref[:]
        wgc = c0_ref[:, 2 * D:2 * D + 32]
        wgs = sh0_ref[:, 2 * D:2 * D + 32]
        s_ua[:, 0:D] = jnp.reshape(
            jnp.dot(wgc, gwc_ref[:], preferred_element_type=f32), (1, D))
        s_ua[:, D:] = jnp.reshape(
            jnp.dot(wgs, gws_ref[:], preferred_element_type=f32), (1, 128))
        s_ba[:, 0:D] = _dnt(gbc_ref[:], wgc) + c0b_ref[:]
        s_ba[:, D:] = _dnt(gbs_ref[:], wgs) + sh0b_ref[:]
        s_wt[0:D, :] = c0_ref[:, :D].astype(bf16)
        s_wt[D:, :] = sh0_ref[:, :D].astype(bf16)
        s_wi[0:D, :] = c0_ref[:, D:2 * D].astype(bf16)
        s_wi[D:, :] = sh0_ref[:, D:2 * D].astype(bf16)
        s_c1[:] = c1_ref[:].astype(bf16)
        s_n1[:] = n1_ref[:].astype(bf16)

    xt = xt_ref[:]
    xi = xi_ref[:]
    g2 = g_ref[:]
    gt = jnp.transpose(g2)
    g = jnp.concatenate(
        [gt[:, r:r + 1] for r in range(g2.shape[0])], axis=0)
    xtb = xt.astype(bf16)
    xib = xi.astype(bf16)
    p_t = _dnt(xtb, s_xtw[:])
    p_i = _dnt(xib, s_xiw[:])
    t_refv = _ln(xt + (p_i[:, :D] + s_bt[:]), lntg_ref[:], lntb_ref[:])
    i_refv = _ln(xi + (p_t[:, :D] + s_bi[:]), lnig_ref[:], lnib_ref[:])
    trb = t_refv.astype(bf16)
    irb = i_refv.astype(bf16)
    q = _dnt(trb, s_wt[:]) + _dnt(irb, s_wi[:]) + g * s_ua[:] + s_ba[:]
    qa = _gelu_exact(q.astype(bf16))
    h0b = qa[:, :D]
    hsb = qa[:, D:]
    h1b = _gelu_exact((_dnt(h0b, s_c1[:]) + c1b_ref[:]).astype(bf16))
    n_pre = p_t[:, D:] + p_i[:, D:] + n0b_ref[:]
    n0b_ = _gelu_exact(n_pre.astype(bf16))
    n1b_ = _gelu_exact((_dnt(n0b_, s_n1[:]) + n1b_ref[:]).astype(bf16))
    conflict = _dnt(h1b, c2_ref[:].astype(bf16)) + c2b_ref[:]
    normal = _dnt(n1b_, m2_ref[:].astype(bf16)) + m2b_ref[:]
    sarcasm = _dnt(hsb, sh1_ref[:].astype(bf16)) + sh1b_ref[:]
    tau = jax.nn.sigmoid(lt_ref[:])
    gate = jax.nn.sigmoid((g - tau) * TEMPERATURE)
    logits_ref[:] = gate * conflict + (1.0 - gate) * normal
    routing_ref[:] = (g2 > tau).astype(f32)
    normal_ref[:] = normal
    conflict_ref[:] = conflict
    sarcasm_ref[:] = sarcasm


@jax.jit
def _run(s_t, s_i, gds, params):
    f32 = jnp.float32
    bf16 = jnp.bfloat16
    p = params
    B = s_t.shape[0]
    grid = (B // BLOCK,)
    row = lambda i: (i, 0)
    rep = lambda i: (0, 0)
    vhalf = lambda i: (1, 0)

    in_specs = [
        pl.BlockSpec((BLOCK, D), row),
        pl.BlockSpec((BLOCK, D), row),
        pl.BlockSpec((BLOCK // 128, 128), row),
        pl.BlockSpec((D, D), vhalf),
        pl.BlockSpec((D, D), rep),
        pl.BlockSpec((D, D), vhalf),
        pl.BlockSpec((D, D), rep),
        pl.BlockSpec((1, D), rep),
        pl.BlockSpec((1, D), rep),
        pl.BlockSpec((1, D), rep),
        pl.BlockSpec((1, D), rep),
        pl.BlockSpec((1, D), rep),
        pl.BlockSpec((1, D), rep),
        pl.BlockSpec((1, D), rep),
        pl.BlockSpec((1, D), rep),
        pl.BlockSpec((D, 2 * D + 32), rep),
        pl.BlockSpec((1, D), rep),
        pl.BlockSpec((32, 1), rep),
        pl.BlockSpec((1, 32), rep),
        pl.BlockSpec((128, 2 * D + 32), rep),
        pl.BlockSpec((1, 128), rep),
        pl.BlockSpec((32, 1), rep),
        pl.BlockSpec((1, 32), rep),
        pl.BlockSpec((128, D), rep),
        pl.BlockSpec((1, 128), rep),
        pl.BlockSpec((D, 2 * D), rep),
        pl.BlockSpec((1, D), rep),
        pl.BlockSpec((128, D), rep),
        pl.BlockSpec((1, 128), rep),
        pl.BlockSpec((3, 128), rep),
        pl.BlockSpec((1, 3), rep),
        pl.BlockSpec((3, 128), rep),
        pl.BlockSpec((1, 3), rep),
        pl.BlockSpec((2, 128), rep),
        pl.BlockSpec((1, 2), rep),
        pl.BlockSpec((1, 1), rep),
    ]
    out_specs = [
        pl.BlockSpec((BLOCK, 3), row),
        pl.BlockSpec((BLOCK // 128, 128), row),
        pl.BlockSpec((BLOCK, 3), row),
        pl.BlockSpec((BLOCK, 3), row),
        pl.BlockSpec((BLOCK, 2), row),
    ]
    out_shape = [
        jax.ShapeDtypeStruct((B, 3), f32),
        jax.ShapeDtypeStruct((B // 128, 128), f32),
        jax.ShapeDtypeStruct((B, 3), f32),
        jax.ShapeDtypeStruct((B, 3), f32),
        jax.ShapeDtypeStruct((B, 2), f32),
    ]
    scratch_shapes = [
        pltpu.VMEM((2 * D, D), bf16),
        pltpu.VMEM((2 * D, D), bf16),
        pltpu.VMEM((1, D), f32),
        pltpu.VMEM((1, D), f32),
        pltpu.VMEM((1, 384), f32),
        pltpu.VMEM((1, 384), f32),
        pltpu.VMEM((384, D), bf16),
        pltpu.VMEM((384, D), bf16),
        pltpu.VMEM((128, D), bf16),
        pltpu.VMEM((128, D), bf16),
    ]
    outs = pl.pallas_call(
        _kernel_body,
        grid=grid,
        in_specs=in_specs,
        out_specs=out_specs,
        out_shape=out_shape,
        scratch_shapes=scratch_shapes,
        compiler_params=pltpu.CompilerParams(
            dimension_semantics=("arbitrary",)),
    )(s_t, s_i, gds.reshape(B // 128, 128),
      p['ca_kvpi_w'], p['ca_opt_w'], p['ca_kvpt_w'], p['ca_opi_w'],
      p['ca_kvpi_b'].reshape(2, D)[1:], p['ca_opt_b'][None],
      p['ca_kvpt_b'].reshape(2, D)[1:], p['ca_opi_b'][None],
      p['ca_lnt_g'][None], p['ca_lnt_b'][None],
      p['ca_lni_g'][None], p['ca_lni_b'][None],
      p['cb_c0_w'], p['cb_c0_b'][None],
      p['cb_gds_w'], p['cb_gds_b'][None],
      p['sh_h0_w'], p['sh_h0_b'][None],
      p['sh_gds_w'], p['sh_gds_b'][None],
      p['cb_c1_w'], p['cb_c1_b'][None],
      p['nb_m0_w'], p['nb_m0_b'][None],
      p['nb_m1_w'], p['nb_m1_b'][None],
      p['cb_c2_w'], p['cb_c2_b'][None],
      p['nb_m2_w'], p['nb_m2_b'][None],
      p['sh_h1_w'], p['sh_h1_b'][None],
      p['log_threshold'].reshape(1, 1))
    logits, routing, normal, conflict, sarcasm = outs
    return logits, routing.reshape(B), normal, conflict, sarcasm


def kernel(s_t, s_i, gds, params):
    return _run(s_t, s_i, gds, params)
